# baseline (speedup 1.0000x reference)
import functools

import jax
import jax.numpy as jnp
from jax import lax
from jax.experimental import pallas as pl
from jax.experimental.pallas import tpu as pltpu

_EPS = 1e-5


def _hswish(v):
    return v * jnp.clip(v + 3.0, 0.0, 6.0) * (1.0 / 6.0)


def _round_up(v, m):
    return (v + m - 1) // m * m


def _fused_kernel(x_ref, we_ref, be_ref, wd_ref, bd_ref, w1_ref, b1_ref,
                  w2_ref, b2_ref, wp_ref, bp_ref, ws_ref, bs_ref, o_ref,
                  p0_ref, p1_ref, p2_ref, p3_ref, p4_ref, y_ref,
                  *, K, H, W, Wp, base, strip):
    R = H * Wp
    Cmid = we_ref.shape[1]
    p = K // 2
    planes = [p0_ref, p1_ref, p2_ref, p3_ref, p4_ref]
    xb = x_ref[0]

    mid = lax.dot_general(xb, we_ref[...], (((0,), (0,)), ((), ())),
                          preferred_element_type=jnp.float32)
    colv = lax.broadcasted_iota(jnp.int32, (R, 1), 0) % Wp
    colmask = (colv >= p) & (colv < p + W)
    mid = jnp.where(colmask, _hswish(mid + be_ref[...]), 0.0)

    P = p0_ref.shape[0]
    midb = mid.astype(jnp.bfloat16)
    for b in range(K):
        d = base - (b - p)
        planes[b][0:d, :] = jnp.zeros((d, Cmid), jnp.bfloat16)
        planes[b][d + R:P, :] = jnp.zeros((P - d - R, Cmid), jnp.bfloat16)
        planes[b][d:d + R, :] = midb

    wrow = [wd_ref[t:t + 1, :] for t in range(K * K)]
    bd = bd_ref[...]
    smask = colmask[0:strip]
    psum = jnp.zeros((1, Cmid), jnp.float32)
    ext = p * Wp
    for s0 in range(0, R, strip):
        acc = None
        for b in range(K):
            chunk = planes[b][s0 + base - ext:s0 + base + strip + ext, :]
            for a in range(K):
                tap = chunk[Wp * a:Wp * a + strip, :] * wrow[a * K + b]
                acc = tap if acc is None else acc + tap
        ys = _hswish(acc.astype(jnp.float32) + bd)
        psum = psum + jnp.sum(jnp.where(smask, ys, 0.0), axis=0, keepdims=True)
        y_ref[s0:s0 + strip, :] = ys.astype(jnp.bfloat16)

    pooled = psum * (1.0 / (H * W))
    h1 = jnp.dot(pooled, w1_ref[...], preferred_element_type=jnp.float32)
    h1 = jnp.clip(h1 + b1_ref[...], 0.0, 6.0)
    sc = jnp.dot(h1, w2_ref[...], preferred_element_type=jnp.float32)
    scale = _hswish(sc + b2_ref[...])

    gated = y_ref[...] * scale.astype(jnp.bfloat16)
    pt = lax.dot_general(wp_ref[...], gated, (((0,), (1,)), ((), ())),
                         preferred_element_type=jnp.float32)
    sct = lax.dot_general(ws_ref[...], xb, (((0,), (0,)), ((), ())),
                          preferred_element_type=jnp.float32)
    bp_col = jnp.transpose(bp_ref[...])
    bs_col = jnp.transpose(bs_ref[...])
    o_ref[0] = _hswish(pt + bp_col) + sct + bs_col


def _fold_pw(w, b, gamma, beta, mean, var):
    g = gamma * lax.rsqrt(var + _EPS)
    return w * g[None, :], ((b - mean) * g + beta).reshape(1, -1)


def kernel(exp_w, exp_b, exp_bn_gamma, exp_bn_beta, exp_bn_mean, exp_bn_var,
           dw_w, dw_b, dw_bn_gamma, dw_bn_beta, dw_bn_mean, dw_bn_var,
           point_w, point_b, point_bn_gamma, point_bn_beta, point_bn_mean,
           point_bn_var, shortcut_w, shortcut_b, shortcut_bn_gamma,
           shortcut_bn_beta, shortcut_bn_mean, shortcut_bn_var,
           se_w1, se_b1, se_w2, se_b2, x):
    N, Cin, H, W = x.shape
    K = dw_w.shape[0]
    Cmid = exp_w.shape[1]
    Cout = point_w.shape[1]
    Cse = se_w1.shape[1]

    we, be = _fold_pw(exp_w, exp_b, exp_bn_gamma, exp_bn_beta, exp_bn_mean,
                      exp_bn_var)
    wp, bp = _fold_pw(point_w, point_b, point_bn_gamma, point_bn_beta,
                      point_bn_mean, point_bn_var)
    ws, bs = _fold_pw(shortcut_w, shortcut_b, shortcut_bn_gamma,
                      shortcut_bn_beta, shortcut_bn_mean, shortcut_bn_var)
    gd = dw_bn_gamma * lax.rsqrt(dw_bn_var + _EPS)
    wd = (dw_w * gd[None, None, :]).reshape(K * K, Cmid).astype(jnp.bfloat16)
    bd = ((dw_b - dw_bn_mean) * gd + dw_bn_beta).reshape(1, Cmid)

    p = K // 2
    Wp = W + 2 * p
    R = H * Wp
    base = _round_up(p * Wp + p, 16)
    P = _round_up(base + R + p * Wp + p, 16)
    strip = 4 * Wp if (R % (4 * Wp) == 0) else Wp

    xp = jnp.pad(x, ((0, 0), (0, 0), (0, 0), (p, p))).reshape(N, Cin, R)

    out = pl.pallas_call(
        functools.partial(_fused_kernel, K=K, H=H, W=W, Wp=Wp, base=base,
                          strip=strip),
        out_shape=jax.ShapeDtypeStruct((N, Cout, R), jnp.float32),
        grid_spec=pltpu.PrefetchScalarGridSpec(
            num_scalar_prefetch=0,
            grid=(N,),
            in_specs=[
                pl.BlockSpec((1, Cin, R), lambda n: (n, 0, 0)),
                pl.BlockSpec((Cin, Cmid), lambda n: (0, 0)),
                pl.BlockSpec((1, Cmid), lambda n: (0, 0)),
                pl.BlockSpec((K * K, Cmid), lambda n: (0, 0)),
                pl.BlockSpec((1, Cmid), lambda n: (0, 0)),
                pl.BlockSpec((Cmid, Cse), lambda n: (0, 0)),
                pl.BlockSpec((1, Cse), lambda n: (0, 0)),
                pl.BlockSpec((Cse, Cmid), lambda n: (0, 0)),
                pl.BlockSpec((1, Cmid), lambda n: (0, 0)),
                pl.BlockSpec((Cmid, Cout), lambda n: (0, 0)),
                pl.BlockSpec((1, Cout), lambda n: (0, 0)),
                pl.BlockSpec((Cin, Cout), lambda n: (0, 0)),
                pl.BlockSpec((1, Cout), lambda n: (0, 0)),
            ],
            out_specs=pl.BlockSpec((1, Cout, R), lambda n: (n, 0, 0)),
            scratch_shapes=(
                [pltpu.VMEM((P, Cmid), jnp.bfloat16) for _ in range(K)]
                + [pltpu.VMEM((R, Cmid), jnp.bfloat16)]),
        ),
        compiler_params=pltpu.CompilerParams(
            dimension_semantics=("parallel",)),
    )(xp, we, be, wd, bd, se_w1,
      se_b1.reshape(1, Cse), se_w2, se_b2.reshape(1, Cmid),
      wp.astype(jnp.bfloat16), bp, ws, bs)

    return out.reshape(N, Cout, H, Wp)[:, :, :, p:p + W]


# speedup vs baseline: 1.0006x; 1.0006x over previous
import functools

import jax
import jax.numpy as jnp
from jax import lax
from jax.experimental import pallas as pl
from jax.experimental.pallas import tpu as pltpu

_EPS = 1e-5


def _hswish(v):
    return v * jnp.clip(v + 3.0, 0.0, 6.0) * (1.0 / 6.0)


def _round_up(v, m):
    return (v + m - 1) // m * m


def _fused_kernel(x_ref, we_ref, be_ref, wd_ref, bd_ref, w1_ref, b1_ref,
                  w2_ref, b2_ref, wp_ref, bp_ref, ws_ref, bs_ref, o_ref,
                  p0_ref, p1_ref, p2_ref, p3_ref, p4_ref, y_ref,
                  *, K, H, W, Wp, base, strip):
    R = H * Wp
    Cmid = we_ref.shape[1]
    p = K // 2
    planes = [p0_ref, p1_ref, p2_ref, p3_ref, p4_ref]
    xb = x_ref[0]

    mid = lax.dot_general(xb, we_ref[...], (((0,), (0,)), ((), ())),
                          preferred_element_type=jnp.float32)
    colv = lax.broadcasted_iota(jnp.int32, (R, 1), 0) % Wp
    colmask = (colv >= p) & (colv < p + W)
    mid = jnp.where(colmask, _hswish(mid + be_ref[...]), 0.0)

    P = p0_ref.shape[0]
    midb = mid.astype(jnp.bfloat16)
    for b in range(K):
        d = base - (b - p)
        planes[b][0:d, :] = jnp.zeros((d, Cmid), jnp.bfloat16)
        planes[b][d + R:P, :] = jnp.zeros((P - d - R, Cmid), jnp.bfloat16)
        planes[b][d:d + R, :] = midb

    wrow = [wd_ref[t:t + 1, :] for t in range(K * K)]
    bd = bd_ref[...]
    smask = colmask[0:strip]
    psum = jnp.zeros((1, Cmid), jnp.float32)
    ext = p * Wp
    for s0 in range(0, R, strip):
        acc = None
        for b in range(K):
            chunk = planes[b][s0 + base - ext:s0 + base + strip + ext, :]
            for a in range(K):
                tap = chunk[Wp * a:Wp * a + strip, :] * wrow[a * K + b]
                acc = tap if acc is None else acc + tap
        ys = _hswish(acc.astype(jnp.float32) + bd)
        psum = psum + jnp.sum(jnp.where(smask, ys, 0.0), axis=0, keepdims=True)
        y_ref[s0:s0 + strip, :] = ys.astype(jnp.bfloat16)

    pooled = psum * (1.0 / (H * W))
    h1 = jnp.dot(pooled, w1_ref[...], preferred_element_type=jnp.float32)
    h1 = jnp.clip(h1 + b1_ref[...], 0.0, 6.0)
    sc = jnp.dot(h1, w2_ref[...], preferred_element_type=jnp.float32)
    scale = _hswish(sc + b2_ref[...])

    gated = y_ref[...] * scale.astype(jnp.bfloat16)
    pt = lax.dot_general(wp_ref[...], gated, (((0,), (1,)), ((), ())),
                         preferred_element_type=jnp.float32)
    sct = lax.dot_general(ws_ref[...], xb, (((0,), (0,)), ((), ())),
                          preferred_element_type=jnp.float32)
    bp_col = jnp.transpose(bp_ref[...])
    bs_col = jnp.transpose(bs_ref[...])
    o_ref[0] = _hswish(pt + bp_col) + sct + bs_col


def _fold_pw(w, b, gamma, beta, mean, var):
    g = gamma * lax.rsqrt(var + _EPS)
    return w * g[None, :], ((b - mean) * g + beta).reshape(1, -1)


def kernel(exp_w, exp_b, exp_bn_gamma, exp_bn_beta, exp_bn_mean, exp_bn_var,
           dw_w, dw_b, dw_bn_gamma, dw_bn_beta, dw_bn_mean, dw_bn_var,
           point_w, point_b, point_bn_gamma, point_bn_beta, point_bn_mean,
           point_bn_var, shortcut_w, shortcut_b, shortcut_bn_gamma,
           shortcut_bn_beta, shortcut_bn_mean, shortcut_bn_var,
           se_w1, se_b1, se_w2, se_b2, x):
    N, Cin, H, W = x.shape
    K = dw_w.shape[0]
    Cmid = exp_w.shape[1]
    Cout = point_w.shape[1]
    Cse = se_w1.shape[1]

    we, be = _fold_pw(exp_w, exp_b, exp_bn_gamma, exp_bn_beta, exp_bn_mean,
                      exp_bn_var)
    wp, bp = _fold_pw(point_w, point_b, point_bn_gamma, point_bn_beta,
                      point_bn_mean, point_bn_var)
    ws, bs = _fold_pw(shortcut_w, shortcut_b, shortcut_bn_gamma,
                      shortcut_bn_beta, shortcut_bn_mean, shortcut_bn_var)
    gd = dw_bn_gamma * lax.rsqrt(dw_bn_var + _EPS)
    wd = (dw_w * gd[None, None, :]).reshape(K * K, Cmid).astype(jnp.bfloat16)
    bd = ((dw_b - dw_bn_mean) * gd + dw_bn_beta).reshape(1, Cmid)

    p = K // 2
    Wp = W + 2 * p
    R = H * Wp
    base = _round_up(p * Wp + p, 16)
    P = _round_up(base + R + p * Wp + p, 16)
    strip = 4 * Wp if (R % (4 * Wp) == 0) else Wp

    xp = jnp.pad(x, ((0, 0), (0, 0), (0, 0), (p, p))).reshape(N, Cin, R)

    out = pl.pallas_call(
        functools.partial(_fused_kernel, K=K, H=H, W=W, Wp=Wp, base=base,
                          strip=strip),
        out_shape=jax.ShapeDtypeStruct((N, Cout, R), jnp.float32),
        grid_spec=pltpu.PrefetchScalarGridSpec(
            num_scalar_prefetch=0,
            grid=(2, N // 2),
            in_specs=[
                pl.BlockSpec((1, Cin, R), lambda i, j: (i * (N // 2) + j, 0, 0)),
                pl.BlockSpec((Cin, Cmid), lambda i, j: (0, 0)),
                pl.BlockSpec((1, Cmid), lambda i, j: (0, 0)),
                pl.BlockSpec((K * K, Cmid), lambda i, j: (0, 0)),
                pl.BlockSpec((1, Cmid), lambda i, j: (0, 0)),
                pl.BlockSpec((Cmid, Cse), lambda i, j: (0, 0)),
                pl.BlockSpec((1, Cse), lambda i, j: (0, 0)),
                pl.BlockSpec((Cse, Cmid), lambda i, j: (0, 0)),
                pl.BlockSpec((1, Cmid), lambda i, j: (0, 0)),
                pl.BlockSpec((Cmid, Cout), lambda i, j: (0, 0)),
                pl.BlockSpec((1, Cout), lambda i, j: (0, 0)),
                pl.BlockSpec((Cin, Cout), lambda i, j: (0, 0)),
                pl.BlockSpec((1, Cout), lambda i, j: (0, 0)),
            ],
            out_specs=pl.BlockSpec((1, Cout, R),
                                   lambda i, j: (i * (N // 2) + j, 0, 0)),
            scratch_shapes=(
                [pltpu.VMEM((P, Cmid), jnp.bfloat16) for _ in range(K)]
                + [pltpu.VMEM((R, Cmid), jnp.bfloat16)]),
        ),
        compiler_params=pltpu.CompilerParams(
            dimension_semantics=("parallel", "arbitrary")),
    )(xp, we, be, wd, bd, se_w1,
      se_b1.reshape(1, Cse), se_w2, se_b2.reshape(1, Cmid),
      wp.astype(jnp.bfloat16), bp, ws, bs)

    return out.reshape(N, Cout, H, Wp)[:, :, :, p:p + W]


# speedup vs baseline: 1.1813x; 1.1806x over previous
import functools

import jax
import jax.numpy as jnp
from jax import lax
from jax.experimental import pallas as pl
from jax.experimental.pallas import tpu as pltpu

_EPS = 1e-5


def _hswish(v):
    return v * jnp.clip(v + 3.0, 0.0, 6.0) * (1.0 / 6.0)


def _round_up(v, m):
    return (v + m - 1) // m * m


def _fused_kernel(x_ref, we_ref, be_ref, wd_ref, bd_ref, w1_ref, b1_ref,
                  w2_ref, b2_ref, wp_ref, bp_ref, ws_ref, bs_ref, o_ref,
                  mp_ref, p0_ref, p1_ref, p2_ref, p3_ref, p4_ref, y_ref,
                  *, K, H, W, Wp, base, strip):
    R = H * Wp
    Cmid = we_ref.shape[1]
    p = K // 2
    planes = [p0_ref, p1_ref, p2_ref, p3_ref, p4_ref]
    xb = x_ref[0]

    mid = lax.dot_general(xb, we_ref[...], (((0,), (0,)), ((), ())),
                          preferred_element_type=jnp.float32)
    mid = _hswish(mid + be_ref[...])

    mp_ref[...] = jnp.zeros((R, Cmid), jnp.float32)
    for h in range(H):
        mp_ref[Wp * h + p:Wp * h + p + W, :] = mid[W * h:W * h + W, :]

    P = p0_ref.shape[0]
    midb = mp_ref[...].astype(jnp.bfloat16)
    for b in range(K):
        d = base - (b - p)
        planes[b][0:d, :] = jnp.zeros((d, Cmid), jnp.bfloat16)
        planes[b][d + R:P, :] = jnp.zeros((P - d - R, Cmid), jnp.bfloat16)
        planes[b][d:d + R, :] = midb

    wrow = [wd_ref[t:t + 1, :] for t in range(K * K)]
    bd = bd_ref[...]
    colv = lax.broadcasted_iota(jnp.int32, (strip, 1), 0) % Wp
    smask = (colv >= p) & (colv < p + W)
    psum = jnp.zeros((1, Cmid), jnp.float32)
    ext = p * Wp
    for s0 in range(0, R, strip):
        acc = None
        for b in range(K):
            chunk = planes[b][s0 + base - ext:s0 + base + strip + ext, :]
            for a in range(K):
                tap = chunk[Wp * a:Wp * a + strip, :] * wrow[a * K + b]
                acc = tap if acc is None else acc + tap
        ys = _hswish(acc.astype(jnp.float32) + bd)
        psum = psum + jnp.sum(jnp.where(smask, ys, 0.0), axis=0, keepdims=True)
        y_ref[s0:s0 + strip, :] = ys.astype(jnp.bfloat16)

    pooled = psum * (1.0 / (H * W))
    h1 = jnp.dot(pooled, w1_ref[...], preferred_element_type=jnp.float32)
    h1 = jnp.clip(h1 + b1_ref[...], 0.0, 6.0)
    sc = jnp.dot(h1, w2_ref[...], preferred_element_type=jnp.float32)
    scale = _hswish(sc + b2_ref[...])

    gated = y_ref[...] * scale.astype(jnp.bfloat16)
    pt = lax.dot_general(wp_ref[...], gated, (((0,), (1,)), ((), ())),
                         preferred_element_type=jnp.float32)
    sct = lax.dot_general(ws_ref[...], xb, (((0,), (0,)), ((), ())),
                          preferred_element_type=jnp.float32)
    ptv = jnp.concatenate(
        [pt[:, Wp * h + p:Wp * h + p + W] for h in range(H)], axis=1)
    bp_col = jnp.transpose(bp_ref[...])
    bs_col = jnp.transpose(bs_ref[...])
    o_ref[0] = _hswish(ptv + bp_col) + sct + bs_col


def _fold_pw(w, b, gamma, beta, mean, var):
    g = gamma * lax.rsqrt(var + _EPS)
    return w * g[None, :], ((b - mean) * g + beta).reshape(1, -1)


def kernel(exp_w, exp_b, exp_bn_gamma, exp_bn_beta, exp_bn_mean, exp_bn_var,
           dw_w, dw_b, dw_bn_gamma, dw_bn_beta, dw_bn_mean, dw_bn_var,
           point_w, point_b, point_bn_gamma, point_bn_beta, point_bn_mean,
           point_bn_var, shortcut_w, shortcut_b, shortcut_bn_gamma,
           shortcut_bn_beta, shortcut_bn_mean, shortcut_bn_var,
           se_w1, se_b1, se_w2, se_b2, x):
    N, Cin, H, W = x.shape
    K = dw_w.shape[0]
    Cmid = exp_w.shape[1]
    Cout = point_w.shape[1]
    Cse = se_w1.shape[1]

    we, be = _fold_pw(exp_w, exp_b, exp_bn_gamma, exp_bn_beta, exp_bn_mean,
                      exp_bn_var)
    wp, bp = _fold_pw(point_w, point_b, point_bn_gamma, point_bn_beta,
                      point_bn_mean, point_bn_var)
    ws, bs = _fold_pw(shortcut_w, shortcut_b, shortcut_bn_gamma,
                      shortcut_bn_beta, shortcut_bn_mean, shortcut_bn_var)
    gd = dw_bn_gamma * lax.rsqrt(dw_bn_var + _EPS)
    wd = (dw_w * gd[None, None, :]).reshape(K * K, Cmid).astype(jnp.bfloat16)
    bd = ((dw_b - dw_bn_mean) * gd + dw_bn_beta).reshape(1, Cmid)

    p = K // 2
    Wp = W + 2 * p
    R = H * Wp
    base = _round_up(p * Wp + p, 16)
    P = _round_up(base + R + p * Wp + p, 16)
    strip = 4 * Wp if (R % (4 * Wp) == 0) else Wp

    xr = x.reshape(N, Cin, H * W)

    out = pl.pallas_call(
        functools.partial(_fused_kernel, K=K, H=H, W=W, Wp=Wp, base=base,
                          strip=strip),
        out_shape=jax.ShapeDtypeStruct((N, Cout, H * W), jnp.float32),
        grid_spec=pltpu.PrefetchScalarGridSpec(
            num_scalar_prefetch=0,
            grid=(2, N // 2),
            in_specs=[
                pl.BlockSpec((1, Cin, H * W),
                             lambda i, j: (i * (N // 2) + j, 0, 0)),
                pl.BlockSpec((Cin, Cmid), lambda i, j: (0, 0)),
                pl.BlockSpec((1, Cmid), lambda i, j: (0, 0)),
                pl.BlockSpec((K * K, Cmid), lambda i, j: (0, 0)),
                pl.BlockSpec((1, Cmid), lambda i, j: (0, 0)),
                pl.BlockSpec((Cmid, Cse), lambda i, j: (0, 0)),
                pl.BlockSpec((1, Cse), lambda i, j: (0, 0)),
                pl.BlockSpec((Cse, Cmid), lambda i, j: (0, 0)),
                pl.BlockSpec((1, Cmid), lambda i, j: (0, 0)),
                pl.BlockSpec((Cmid, Cout), lambda i, j: (0, 0)),
                pl.BlockSpec((1, Cout), lambda i, j: (0, 0)),
                pl.BlockSpec((Cin, Cout), lambda i, j: (0, 0)),
                pl.BlockSpec((1, Cout), lambda i, j: (0, 0)),
            ],
            out_specs=pl.BlockSpec((1, Cout, H * W),
                                   lambda i, j: (i * (N // 2) + j, 0, 0)),
            scratch_shapes=(
                [pltpu.VMEM((R, Cmid), jnp.float32)]
                + [pltpu.VMEM((P, Cmid), jnp.bfloat16) for _ in range(K)]
                + [pltpu.VMEM((R, Cmid), jnp.bfloat16)]),
        ),
        compiler_params=pltpu.CompilerParams(
            dimension_semantics=("parallel", "arbitrary")),
    )(xr, we, be, wd, bd, se_w1,
      se_b1.reshape(1, Cse), se_w2, se_b2.reshape(1, Cmid),
      wp.astype(jnp.bfloat16), bp, ws, bs)

    return out.reshape(N, Cout, H, W)


# speedup vs baseline: 1.2080x; 1.0226x over previous
import functools

import jax
import jax.numpy as jnp
from jax import lax
from jax.experimental import pallas as pl
from jax.experimental.pallas import tpu as pltpu

_EPS = 1e-5


def _hswish(v):
    return v * jnp.clip(v + 3.0, 0.0, 6.0) * (1.0 / 6.0)


def _round_up(v, m):
    return (v + m - 1) // m * m


def _fused_kernel(x_ref, ew_ref, eb_ref, eg_ref, ebt_ref, em_ref, ev_ref,
                  dw_ref, db_ref, dg_ref, dbt_ref, dm_ref, dv_ref,
                  pw_ref, pb_ref, pg_ref, pbt_ref, pm_ref, pv_ref,
                  sw_ref, sb_ref, sg_ref, sbt_ref, sm_ref, sv_ref,
                  w1_ref, b1_ref, w2_ref, b2_ref, o_ref,
                  mp_ref, p0_ref, p1_ref, p2_ref, p3_ref, p4_ref, y_ref,
                  *, K, H, W, Wp, base, strip):
    R = H * Wp
    Cmid = ew_ref.shape[1]
    p = K // 2
    planes = [p0_ref, p1_ref, p2_ref, p3_ref, p4_ref]
    xb = x_ref[0]

    ge = eg_ref[...] * lax.rsqrt(ev_ref[...] + _EPS)
    be = (eb_ref[...] - em_ref[...]) * ge + ebt_ref[...]
    gd = dg_ref[...] * lax.rsqrt(dv_ref[...] + _EPS)
    bd = (db_ref[...] - dm_ref[...]) * gd + dbt_ref[...]
    gp = pg_ref[...] * lax.rsqrt(pv_ref[...] + _EPS)
    bp = (pb_ref[...] - pm_ref[...]) * gp + pbt_ref[...]
    gs = sg_ref[...] * lax.rsqrt(sv_ref[...] + _EPS)
    bs = (sb_ref[...] - sm_ref[...]) * gs + sbt_ref[...]
    we = ew_ref[...] * ge
    wd = (dw_ref[...] * gd).astype(jnp.bfloat16)
    wp = (pw_ref[...] * gp).astype(jnp.bfloat16)
    ws = sw_ref[...] * gs

    mid = lax.dot_general(xb, we, (((0,), (0,)), ((), ())),
                          preferred_element_type=jnp.float32)
    mid = _hswish(mid + be)

    mp_ref[...] = jnp.zeros((R, Cmid), jnp.float32)
    for h in range(H):
        mp_ref[Wp * h + p:Wp * h + p + W, :] = mid[W * h:W * h + W, :]

    P = p0_ref.shape[0]
    midb = mp_ref[...].astype(jnp.bfloat16)
    for b in range(K):
        d = base - (b - p)
        planes[b][0:d, :] = jnp.zeros((d, Cmid), jnp.bfloat16)
        planes[b][d + R:P, :] = jnp.zeros((P - d - R, Cmid), jnp.bfloat16)
        planes[b][d:d + R, :] = midb

    wrow = [wd[t:t + 1, :] for t in range(K * K)]
    colv = lax.broadcasted_iota(jnp.int32, (strip, 1), 0) % Wp
    smask = (colv >= p) & (colv < p + W)
    psum = jnp.zeros((1, Cmid), jnp.float32)
    ext = p * Wp
    for s0 in range(0, R, strip):
        acc = None
        for b in range(K):
            chunk = planes[b][s0 + base - ext:s0 + base + strip + ext, :]
            for a in range(K):
                tap = chunk[Wp * a:Wp * a + strip, :] * wrow[a * K + b]
                acc = tap if acc is None else acc + tap
        ys = _hswish(acc.astype(jnp.float32) + bd)
        psum = psum + jnp.sum(jnp.where(smask, ys, 0.0), axis=0, keepdims=True)
        y_ref[s0:s0 + strip, :] = ys.astype(jnp.bfloat16)

    pooled = psum * (1.0 / (H * W))
    h1 = jnp.dot(pooled, w1_ref[...], preferred_element_type=jnp.float32)
    h1 = jnp.clip(h1 + b1_ref[...], 0.0, 6.0)
    sc = jnp.dot(h1, w2_ref[...], preferred_element_type=jnp.float32)
    scale = _hswish(sc + b2_ref[...])

    gated = y_ref[...] * scale.astype(jnp.bfloat16)
    pt = lax.dot_general(wp, gated, (((0,), (1,)), ((), ())),
                         preferred_element_type=jnp.float32)
    sct = lax.dot_general(ws, xb, (((0,), (0,)), ((), ())),
                          preferred_element_type=jnp.float32)
    ptv = jnp.concatenate(
        [pt[:, Wp * h + p:Wp * h + p + W] for h in range(H)], axis=1)
    bp_col = jnp.transpose(bp)
    bs_col = jnp.transpose(bs)
    o_ref[0] = _hswish(ptv + bp_col) + sct + bs_col


def kernel(exp_w, exp_b, exp_bn_gamma, exp_bn_beta, exp_bn_mean, exp_bn_var,
           dw_w, dw_b, dw_bn_gamma, dw_bn_beta, dw_bn_mean, dw_bn_var,
           point_w, point_b, point_bn_gamma, point_bn_beta, point_bn_mean,
           point_bn_var, shortcut_w, shortcut_b, shortcut_bn_gamma,
           shortcut_bn_beta, shortcut_bn_mean, shortcut_bn_var,
           se_w1, se_b1, se_w2, se_b2, x):
    N, Cin, H, W = x.shape
    K = dw_w.shape[0]
    Cmid = exp_w.shape[1]
    Cout = point_w.shape[1]
    Cse = se_w1.shape[1]

    p = K // 2
    Wp = W + 2 * p
    R = H * Wp
    base = _round_up(p * Wp + p, 16)
    P = _round_up(base + R + p * Wp + p, 16)
    strip = 4 * Wp if (R % (4 * Wp) == 0) else Wp

    xr = x.reshape(N, Cin, H * W)
    row = lambda v: v.reshape(1, -1)

    def bspec(*shape):
        return pl.BlockSpec(shape, lambda i, j: (0,) * len(shape))

    in_specs = [pl.BlockSpec((1, Cin, H * W),
                             lambda i, j: (i * (N // 2) + j, 0, 0))]
    in_specs += [bspec(Cin, Cmid)] + [bspec(1, Cmid)] * 5
    in_specs += [bspec(K * K, Cmid)] + [bspec(1, Cmid)] * 5
    in_specs += [bspec(Cmid, Cout)] + [bspec(1, Cout)] * 5
    in_specs += [bspec(Cin, Cout)] + [bspec(1, Cout)] * 5
    in_specs += [bspec(Cmid, Cse), bspec(1, Cse), bspec(Cse, Cmid),
                 bspec(1, Cmid)]

    out = pl.pallas_call(
        functools.partial(_fused_kernel, K=K, H=H, W=W, Wp=Wp, base=base,
                          strip=strip),
        out_shape=jax.ShapeDtypeStruct((N, Cout, H * W), jnp.float32),
        grid_spec=pltpu.PrefetchScalarGridSpec(
            num_scalar_prefetch=0,
            grid=(2, N // 2),
            in_specs=in_specs,
            out_specs=pl.BlockSpec((1, Cout, H * W),
                                   lambda i, j: (i * (N // 2) + j, 0, 0)),
            scratch_shapes=(
                [pltpu.VMEM((R, Cmid), jnp.float32)]
                + [pltpu.VMEM((P, Cmid), jnp.bfloat16) for _ in range(K)]
                + [pltpu.VMEM((R, Cmid), jnp.bfloat16)]),
        ),
        compiler_params=pltpu.CompilerParams(
            dimension_semantics=("parallel", "arbitrary")),
    )(xr,
      exp_w, row(exp_b), row(exp_bn_gamma), row(exp_bn_beta),
      row(exp_bn_mean), row(exp_bn_var),
      dw_w.reshape(K * K, Cmid), row(dw_b), row(dw_bn_gamma),
      row(dw_bn_beta), row(dw_bn_mean), row(dw_bn_var),
      point_w, row(point_b), row(point_bn_gamma), row(point_bn_beta),
      row(point_bn_mean), row(point_bn_var),
      shortcut_w, row(shortcut_b), row(shortcut_bn_gamma),
      row(shortcut_bn_beta), row(shortcut_bn_mean), row(shortcut_bn_var),
      se_w1, row(se_b1), se_w2, row(se_b2))

    return out.reshape(N, Cout, H, W)


# speedup vs baseline: 1.2553x; 1.0391x over previous
import functools

import jax
import jax.numpy as jnp
from jax import lax
from jax.experimental import pallas as pl
from jax.experimental.pallas import tpu as pltpu

_EPS = 1e-5


def _hswish(v):
    return v * jnp.clip(v + 3.0, 0.0, 6.0) * (1.0 / 6.0)


def _round_up(v, m):
    return (v + m - 1) // m * m


def _fused_kernel(x_ref, ew_ref, eb_ref, eg_ref, ebt_ref, em_ref, ev_ref,
                  dw_ref, db_ref, dg_ref, dbt_ref, dm_ref, dv_ref,
                  pw_ref, pb_ref, pg_ref, pbt_ref, pm_ref, pv_ref,
                  sw_ref, sb_ref, sg_ref, sbt_ref, sm_ref, sv_ref,
                  w1_ref, b1_ref, w2_ref, b2_ref, o_ref,
                  mp_ref, p0_ref, p1_ref, p2_ref, p3_ref, p4_ref, y_ref,
                  *, K, H, W, Wp, base, strip, BB):
    R = H * Wp
    Cmid = ew_ref.shape[1]
    p = K // 2
    planes = [p0_ref, p1_ref, p2_ref, p3_ref, p4_ref]

    ge = eg_ref[...] * lax.rsqrt(ev_ref[...] + _EPS)
    be = (eb_ref[...] - em_ref[...]) * ge + ebt_ref[...]
    gd = dg_ref[...] * lax.rsqrt(dv_ref[...] + _EPS)
    bd = (db_ref[...] - dm_ref[...]) * gd + dbt_ref[...]
    gp = pg_ref[...] * lax.rsqrt(pv_ref[...] + _EPS)
    bp = (pb_ref[...] - pm_ref[...]) * gp + pbt_ref[...]
    gs = sg_ref[...] * lax.rsqrt(sv_ref[...] + _EPS)
    bs = (sb_ref[...] - sm_ref[...]) * gs + sbt_ref[...]
    we = ew_ref[...] * ge
    wd = (dw_ref[...] * gd).astype(jnp.bfloat16)
    wp = (pw_ref[...] * gp).astype(jnp.bfloat16)
    ws = sw_ref[...] * gs

    P = p0_ref.shape[0]
    wrow = [wd[t:t + 1, :] for t in range(K * K)]
    colv = lax.broadcasted_iota(jnp.int32, (strip, 1), 0) % Wp
    smask = (colv >= p) & (colv < p + W)
    ext = p * Wp
    bp_col = jnp.transpose(bp)
    bs_col = jnp.transpose(bs)

    for n in range(BB):
        xb = x_ref[n]

        mid = lax.dot_general(xb, we, (((0,), (0,)), ((), ())),
                              preferred_element_type=jnp.float32)
        mid = _hswish(mid + be)

        mp_ref[...] = jnp.zeros((R, Cmid), jnp.float32)
        for h in range(H):
            mp_ref[Wp * h + p:Wp * h + p + W, :] = mid[W * h:W * h + W, :]

        midb = mp_ref[...].astype(jnp.bfloat16)
        for b in range(K):
            d = base - (b - p)
            planes[b][0:d, :] = jnp.zeros((d, Cmid), jnp.bfloat16)
            planes[b][d + R:P, :] = jnp.zeros((P - d - R, Cmid), jnp.bfloat16)
            planes[b][d:d + R, :] = midb

        psum = jnp.zeros((1, Cmid), jnp.float32)
        for s0 in range(0, R, strip):
            acc = None
            for b in range(K):
                chunk = planes[b][s0 + base - ext:s0 + base + strip + ext, :]
                for a in range(K):
                    tap = chunk[Wp * a:Wp * a + strip, :] * wrow[a * K + b]
                    acc = tap if acc is None else acc + tap
            ys = _hswish(acc.astype(jnp.float32) + bd)
            psum = psum + jnp.sum(jnp.where(smask, ys, 0.0), axis=0,
                                  keepdims=True)
            y_ref[s0:s0 + strip, :] = ys.astype(jnp.bfloat16)

        pooled = psum * (1.0 / (H * W))
        h1 = jnp.dot(pooled, w1_ref[...], preferred_element_type=jnp.float32)
        h1 = jnp.clip(h1 + b1_ref[...], 0.0, 6.0)
        sc = jnp.dot(h1, w2_ref[...], preferred_element_type=jnp.float32)
        scale = _hswish(sc + b2_ref[...])

        gated = y_ref[...] * scale.astype(jnp.bfloat16)
        pt = lax.dot_general(wp, gated, (((0,), (1,)), ((), ())),
                             preferred_element_type=jnp.float32)
        sct = lax.dot_general(ws, xb, (((0,), (0,)), ((), ())),
                              preferred_element_type=jnp.float32)
        ptv = jnp.concatenate(
            [pt[:, Wp * h + p:Wp * h + p + W] for h in range(H)], axis=1)
        o_ref[n] = _hswish(ptv + bp_col) + sct + bs_col


def kernel(exp_w, exp_b, exp_bn_gamma, exp_bn_beta, exp_bn_mean, exp_bn_var,
           dw_w, dw_b, dw_bn_gamma, dw_bn_beta, dw_bn_mean, dw_bn_var,
           point_w, point_b, point_bn_gamma, point_bn_beta, point_bn_mean,
           point_bn_var, shortcut_w, shortcut_b, shortcut_bn_gamma,
           shortcut_bn_beta, shortcut_bn_mean, shortcut_bn_var,
           se_w1, se_b1, se_w2, se_b2, x):
    N, Cin, H, W = x.shape
    K = dw_w.shape[0]
    Cmid = exp_w.shape[1]
    Cout = point_w.shape[1]
    Cse = se_w1.shape[1]

    p = K // 2
    Wp = W + 2 * p
    R = H * Wp
    base = _round_up(p * Wp + p, 16)
    P = _round_up(base + R + p * Wp + p, 16)
    strip = 4 * Wp if (R % (4 * Wp) == 0) else Wp

    xr = x.reshape(N, Cin, H * W)
    row = lambda v: v.reshape(1, -1)

    def bspec(*shape):
        return pl.BlockSpec(shape, lambda i, j: (0,) * len(shape))

    BB = 2 if N % 4 == 0 else 1
    nb = N // BB
    in_specs = [pl.BlockSpec((BB, Cin, H * W),
                             lambda i, j: (i * (nb // 2) + j, 0, 0))]
    in_specs += [bspec(Cin, Cmid)] + [bspec(1, Cmid)] * 5
    in_specs += [bspec(K * K, Cmid)] + [bspec(1, Cmid)] * 5
    in_specs += [bspec(Cmid, Cout)] + [bspec(1, Cout)] * 5
    in_specs += [bspec(Cin, Cout)] + [bspec(1, Cout)] * 5
    in_specs += [bspec(Cmid, Cse), bspec(1, Cse), bspec(Cse, Cmid),
                 bspec(1, Cmid)]

    out = pl.pallas_call(
        functools.partial(_fused_kernel, K=K, H=H, W=W, Wp=Wp, base=base,
                          strip=strip, BB=BB),
        out_shape=jax.ShapeDtypeStruct((N, Cout, H * W), jnp.float32),
        grid_spec=pltpu.PrefetchScalarGridSpec(
            num_scalar_prefetch=0,
            grid=(2, nb // 2),
            in_specs=in_specs,
            out_specs=pl.BlockSpec((BB, Cout, H * W),
                                   lambda i, j: (i * (nb // 2) + j, 0, 0)),
            scratch_shapes=(
                [pltpu.VMEM((R, Cmid), jnp.float32)]
                + [pltpu.VMEM((P, Cmid), jnp.bfloat16) for _ in range(K)]
                + [pltpu.VMEM((R, Cmid), jnp.bfloat16)]),
        ),
        compiler_params=pltpu.CompilerParams(
            dimension_semantics=("parallel", "arbitrary")),
    )(xr,
      exp_w, row(exp_b), row(exp_bn_gamma), row(exp_bn_beta),
      row(exp_bn_mean), row(exp_bn_var),
      dw_w.reshape(K * K, Cmid), row(dw_b), row(dw_bn_gamma),
      row(dw_bn_beta), row(dw_bn_mean), row(dw_bn_var),
      point_w, row(point_b), row(point_bn_gamma), row(point_bn_beta),
      row(point_bn_mean), row(point_bn_var),
      shortcut_w, row(shortcut_b), row(shortcut_bn_gamma),
      row(shortcut_bn_beta), row(shortcut_bn_mean), row(shortcut_bn_var),
      se_w1, row(se_b1), se_w2, row(se_b2))

    return out.reshape(N, Cout, H, W)


# speedup vs baseline: 1.2734x; 1.0144x over previous
import functools

import jax
import jax.numpy as jnp
from jax import lax
from jax.experimental import pallas as pl
from jax.experimental.pallas import tpu as pltpu

_EPS = 1e-5


def _hswish(v):
    return v * jnp.clip(v + 3.0, 0.0, 6.0) * (1.0 / 6.0)


def _round_up(v, m):
    return (v + m - 1) // m * m


def _fused_kernel(x_ref, ew_ref, eb_ref, eg_ref, ebt_ref, em_ref, ev_ref,
                  dw_ref, db_ref, dg_ref, dbt_ref, dm_ref, dv_ref,
                  pw_ref, pb_ref, pg_ref, pbt_ref, pm_ref, pv_ref,
                  sw_ref, sb_ref, sg_ref, sbt_ref, sm_ref, sv_ref,
                  w1_ref, b1_ref, w2_ref, b2_ref, o_ref,
                  mp_ref, p0_ref, p1_ref, p2_ref, p3_ref, p4_ref, y_ref,
                  *, K, H, W, Wp, base, strip, BB):
    R = H * Wp
    Cmid = ew_ref.shape[1]
    p = K // 2
    planes = [p0_ref, p1_ref, p2_ref, p3_ref, p4_ref]

    ge = eg_ref[...] * lax.rsqrt(ev_ref[...] + _EPS)
    be = (eb_ref[...] - em_ref[...]) * ge + ebt_ref[...]
    gd = dg_ref[...] * lax.rsqrt(dv_ref[...] + _EPS)
    bd = (db_ref[...] - dm_ref[...]) * gd + dbt_ref[...]
    gp = pg_ref[...] * lax.rsqrt(pv_ref[...] + _EPS)
    bp = (pb_ref[...] - pm_ref[...]) * gp + pbt_ref[...]
    gs = sg_ref[...] * lax.rsqrt(sv_ref[...] + _EPS)
    bs = (sb_ref[...] - sm_ref[...]) * gs + sbt_ref[...]
    we = ew_ref[...] * ge
    wd = (dw_ref[...] * gd).astype(jnp.bfloat16)
    wp = (pw_ref[...] * gp).astype(jnp.bfloat16)
    ws = sw_ref[...] * gs

    P = p0_ref.shape[0]
    wrow = [wd[t:t + 1, :] for t in range(K * K)]
    colv = lax.broadcasted_iota(jnp.int32, (strip, 1), 0) % Wp
    smask = (colv >= p) & (colv < p + W)
    ext = p * Wp
    bp_col = jnp.transpose(bp)
    bs_col = jnp.transpose(bs)

    for n in range(BB):
        xb = x_ref[n]

        mid = lax.dot_general(xb, we, (((0,), (0,)), ((), ())),
                              preferred_element_type=jnp.float32)
        mid = _hswish(mid + be)

        mp_ref[...] = jnp.zeros((R, Cmid), jnp.float32)
        for h in range(H):
            mp_ref[Wp * h + p:Wp * h + p + W, :] = mid[W * h:W * h + W, :]

        midb = mp_ref[...].astype(jnp.bfloat16)
        for b in range(K):
            d = base - (b - p)
            planes[b][0:d, :] = jnp.zeros((d, Cmid), jnp.bfloat16)
            planes[b][d + R:P, :] = jnp.zeros((P - d - R, Cmid), jnp.bfloat16)
            planes[b][d:d + R, :] = midb

        psum = jnp.zeros((1, Cmid), jnp.float32)
        for s0 in range(0, R, strip):
            acc = None
            for b in range(K):
                chunk = planes[b][s0 + base - ext:s0 + base + strip + ext, :]
                for a in range(K):
                    tap = chunk[Wp * a:Wp * a + strip, :] * wrow[a * K + b]
                    acc = tap if acc is None else acc + tap
            ys = _hswish(acc.astype(jnp.float32) + bd)
            psum = psum + jnp.sum(jnp.where(smask, ys, 0.0), axis=0,
                                  keepdims=True)
            y_ref[s0:s0 + strip, :] = ys.astype(jnp.bfloat16)

        pooled = psum * (1.0 / (H * W))
        h1 = jnp.dot(pooled, w1_ref[...], preferred_element_type=jnp.float32)
        h1 = jnp.clip(h1 + b1_ref[...], 0.0, 6.0)
        sc = jnp.dot(h1, w2_ref[...], preferred_element_type=jnp.float32)
        scale = _hswish(sc + b2_ref[...])

        gated = y_ref[...] * scale.astype(jnp.bfloat16)
        pt = lax.dot_general(wp, gated, (((0,), (1,)), ((), ())),
                             preferred_element_type=jnp.float32)
        sct = lax.dot_general(ws, xb, (((0,), (0,)), ((), ())),
                              preferred_element_type=jnp.float32)
        ptv = jnp.concatenate(
            [pt[:, Wp * h + p:Wp * h + p + W] for h in range(H)], axis=1)
        o_ref[n] = _hswish(ptv + bp_col) + sct + bs_col


def kernel(exp_w, exp_b, exp_bn_gamma, exp_bn_beta, exp_bn_mean, exp_bn_var,
           dw_w, dw_b, dw_bn_gamma, dw_bn_beta, dw_bn_mean, dw_bn_var,
           point_w, point_b, point_bn_gamma, point_bn_beta, point_bn_mean,
           point_bn_var, shortcut_w, shortcut_b, shortcut_bn_gamma,
           shortcut_bn_beta, shortcut_bn_mean, shortcut_bn_var,
           se_w1, se_b1, se_w2, se_b2, x):
    N, Cin, H, W = x.shape
    K = dw_w.shape[0]
    Cmid = exp_w.shape[1]
    Cout = point_w.shape[1]
    Cse = se_w1.shape[1]

    p = K // 2
    Wp = W + 2 * p
    R = H * Wp
    base = _round_up(p * Wp + p, 16)
    P = _round_up(base + R + p * Wp + p, 16)
    strip = 4 * Wp if (R % (4 * Wp) == 0) else Wp

    xr = x.reshape(N, Cin, H * W)
    row = lambda v: v.reshape(1, -1)

    def bspec(*shape):
        return pl.BlockSpec(shape, lambda i, j: (0,) * len(shape))

    BB = 4 if N % 8 == 0 else (2 if N % 4 == 0 else 1)
    nb = N // BB
    in_specs = [pl.BlockSpec((BB, Cin, H * W),
                             lambda i, j: (i * (nb // 2) + j, 0, 0))]
    in_specs += [bspec(Cin, Cmid)] + [bspec(1, Cmid)] * 5
    in_specs += [bspec(K * K, Cmid)] + [bspec(1, Cmid)] * 5
    in_specs += [bspec(Cmid, Cout)] + [bspec(1, Cout)] * 5
    in_specs += [bspec(Cin, Cout)] + [bspec(1, Cout)] * 5
    in_specs += [bspec(Cmid, Cse), bspec(1, Cse), bspec(Cse, Cmid),
                 bspec(1, Cmid)]

    out = pl.pallas_call(
        functools.partial(_fused_kernel, K=K, H=H, W=W, Wp=Wp, base=base,
                          strip=strip, BB=BB),
        out_shape=jax.ShapeDtypeStruct((N, Cout, H * W), jnp.float32),
        grid_spec=pltpu.PrefetchScalarGridSpec(
            num_scalar_prefetch=0,
            grid=(2, nb // 2),
            in_specs=in_specs,
            out_specs=pl.BlockSpec((BB, Cout, H * W),
                                   lambda i, j: (i * (nb // 2) + j, 0, 0)),
            scratch_shapes=(
                [pltpu.VMEM((R, Cmid), jnp.float32)]
                + [pltpu.VMEM((P, Cmid), jnp.bfloat16) for _ in range(K)]
                + [pltpu.VMEM((R, Cmid), jnp.bfloat16)]),
        ),
        compiler_params=pltpu.CompilerParams(
            dimension_semantics=("parallel", "arbitrary")),
    )(xr,
      exp_w, row(exp_b), row(exp_bn_gamma), row(exp_bn_beta),
      row(exp_bn_mean), row(exp_bn_var),
      dw_w.reshape(K * K, Cmid), row(dw_b), row(dw_bn_gamma),
      row(dw_bn_beta), row(dw_bn_mean), row(dw_bn_var),
      point_w, row(point_b), row(point_bn_gamma), row(point_bn_beta),
      row(point_bn_mean), row(point_bn_var),
      shortcut_w, row(shortcut_b), row(shortcut_bn_gamma),
      row(shortcut_bn_beta), row(shortcut_bn_mean), row(shortcut_bn_var),
      se_w1, row(se_b1), se_w2, row(se_b2))

    return out.reshape(N, Cout, H, W)


# speedup vs baseline: 1.3168x; 1.0341x over previous
import functools

import jax
import jax.numpy as jnp
from jax import lax
from jax.experimental import pallas as pl
from jax.experimental.pallas import tpu as pltpu

_EPS = 1e-5


def _hswish(v):
    return v * jnp.clip(v + 3.0, 0.0, 6.0) * (1.0 / 6.0)


def _round_up(v, m):
    return (v + m - 1) // m * m


def _fused_kernel(x_ref, ew_ref, eb_ref, eg_ref, ebt_ref, em_ref, ev_ref,
                  dw_ref, db_ref, dg_ref, dbt_ref, dm_ref, dv_ref,
                  pw_ref, pb_ref, pg_ref, pbt_ref, pm_ref, pv_ref,
                  sw_ref, sb_ref, sg_ref, sbt_ref, sm_ref, sv_ref,
                  w1_ref, b1_ref, w2_ref, b2_ref, o_ref,
                  mp_ref, p0_ref, p1_ref, p2_ref, p3_ref, p4_ref, y_ref,
                  *, K, H, W, Wp, base, strip, BB):
    R = H * Wp
    Cmid = ew_ref.shape[1]
    p = K // 2
    planes = [p0_ref, p1_ref, p2_ref, p3_ref, p4_ref]

    ge = eg_ref[...] * lax.rsqrt(ev_ref[...] + _EPS)
    be = (eb_ref[...] - em_ref[...]) * ge + ebt_ref[...]
    gd = dg_ref[...] * lax.rsqrt(dv_ref[...] + _EPS)
    bd = (db_ref[...] - dm_ref[...]) * gd + dbt_ref[...]
    gp = pg_ref[...] * lax.rsqrt(pv_ref[...] + _EPS)
    bp = (pb_ref[...] - pm_ref[...]) * gp + pbt_ref[...]
    gs = sg_ref[...] * lax.rsqrt(sv_ref[...] + _EPS)
    bs = (sb_ref[...] - sm_ref[...]) * gs + sbt_ref[...]
    we = ew_ref[...] * ge
    wd = (dw_ref[...] * gd).astype(jnp.bfloat16)
    wp = (pw_ref[...] * gp).astype(jnp.bfloat16)
    ws = sw_ref[...] * gs

    P = p0_ref.shape[0]
    wrow = [wd[t:t + 1, :] for t in range(K * K)]
    colv = lax.broadcasted_iota(jnp.int32, (strip, 1), 0) % Wp
    smask = colv < W
    ext = p * Wp
    bp_col = jnp.transpose(bp)
    bs_col = jnp.transpose(bs)

    for n in range(BB):
        xb = x_ref[n]

        mid = lax.dot_general(xb, we, (((0,), (0,)), ((), ())),
                              preferred_element_type=jnp.float32)
        mid = _hswish(mid + be)

        mp_ref[...] = jnp.zeros((R, Cmid), jnp.float32)
        for h in range(H):
            mp_ref[Wp * h:Wp * h + W, :] = mid[W * h:W * h + W, :]

        midb = mp_ref[...].astype(jnp.bfloat16)
        for b in range(K):
            d = base - (b - p)
            planes[b][0:d, :] = jnp.zeros((d, Cmid), jnp.bfloat16)
            planes[b][d + R:P, :] = jnp.zeros((P - d - R, Cmid), jnp.bfloat16)
            planes[b][d:d + R, :] = midb

        psum = jnp.zeros((1, Cmid), jnp.float32)
        for s0 in range(0, R, strip):
            acc = None
            for b in range(K):
                chunk = planes[b][s0 + base - ext:s0 + base + strip + ext, :]
                for a in range(K):
                    tap = chunk[Wp * a:Wp * a + strip, :] * wrow[a * K + b]
                    acc = tap if acc is None else acc + tap
            ys = _hswish(acc.astype(jnp.float32) + bd)
            psum = psum + jnp.sum(jnp.where(smask, ys, 0.0), axis=0,
                                  keepdims=True)
            y_ref[s0:s0 + strip, :] = ys.astype(jnp.bfloat16)

        pooled = psum * (1.0 / (H * W))
        h1 = jnp.dot(pooled, w1_ref[...], preferred_element_type=jnp.float32)
        h1 = jnp.clip(h1 + b1_ref[...], 0.0, 6.0)
        sc = jnp.dot(h1, w2_ref[...], preferred_element_type=jnp.float32)
        scale = _hswish(sc + b2_ref[...])

        gated = y_ref[...] * scale.astype(jnp.bfloat16)
        pt = lax.dot_general(wp, gated, (((0,), (1,)), ((), ())),
                             preferred_element_type=jnp.float32)
        sct = lax.dot_general(ws, xb, (((0,), (0,)), ((), ())),
                              preferred_element_type=jnp.float32)
        ptv = jnp.concatenate(
            [pt[:, Wp * h:Wp * h + W] for h in range(H)], axis=1)
        o_ref[n] = _hswish(ptv + bp_col) + sct + bs_col


def kernel(exp_w, exp_b, exp_bn_gamma, exp_bn_beta, exp_bn_mean, exp_bn_var,
           dw_w, dw_b, dw_bn_gamma, dw_bn_beta, dw_bn_mean, dw_bn_var,
           point_w, point_b, point_bn_gamma, point_bn_beta, point_bn_mean,
           point_bn_var, shortcut_w, shortcut_b, shortcut_bn_gamma,
           shortcut_bn_beta, shortcut_bn_mean, shortcut_bn_var,
           se_w1, se_b1, se_w2, se_b2, x):
    N, Cin, H, W = x.shape
    K = dw_w.shape[0]
    Cmid = exp_w.shape[1]
    Cout = point_w.shape[1]
    Cse = se_w1.shape[1]

    p = K // 2
    Wp = W + 2 * p
    R = H * Wp
    base = _round_up(p * Wp + p, 16)
    P = _round_up(base + R + p * Wp + p, 16)
    strip = 4 * Wp if (R % (4 * Wp) == 0) else Wp

    xr = x.reshape(N, Cin, H * W)
    row = lambda v: v.reshape(1, -1)

    def bspec(*shape):
        return pl.BlockSpec(shape, lambda i, j: (0,) * len(shape))

    BB = 4 if N % 8 == 0 else (2 if N % 4 == 0 else 1)
    nb = N // BB
    in_specs = [pl.BlockSpec((BB, Cin, H * W),
                             lambda i, j: (i * (nb // 2) + j, 0, 0))]
    in_specs += [bspec(Cin, Cmid)] + [bspec(1, Cmid)] * 5
    in_specs += [bspec(K * K, Cmid)] + [bspec(1, Cmid)] * 5
    in_specs += [bspec(Cmid, Cout)] + [bspec(1, Cout)] * 5
    in_specs += [bspec(Cin, Cout)] + [bspec(1, Cout)] * 5
    in_specs += [bspec(Cmid, Cse), bspec(1, Cse), bspec(Cse, Cmid),
                 bspec(1, Cmid)]

    out = pl.pallas_call(
        functools.partial(_fused_kernel, K=K, H=H, W=W, Wp=Wp, base=base,
                          strip=strip, BB=BB),
        out_shape=jax.ShapeDtypeStruct((N, Cout, H * W), jnp.float32),
        grid_spec=pltpu.PrefetchScalarGridSpec(
            num_scalar_prefetch=0,
            grid=(2, nb // 2),
            in_specs=in_specs,
            out_specs=pl.BlockSpec((BB, Cout, H * W),
                                   lambda i, j: (i * (nb // 2) + j, 0, 0)),
            scratch_shapes=(
                [pltpu.VMEM((R, Cmid), jnp.float32)]
                + [pltpu.VMEM((P, Cmid), jnp.bfloat16) for _ in range(K)]
                + [pltpu.VMEM((R, Cmid), jnp.bfloat16)]),
        ),
        compiler_params=pltpu.CompilerParams(
            dimension_semantics=("parallel", "arbitrary")),
    )(xr,
      exp_w, row(exp_b), row(exp_bn_gamma), row(exp_bn_beta),
      row(exp_bn_mean), row(exp_bn_var),
      dw_w.reshape(K * K, Cmid), row(dw_b), row(dw_bn_gamma),
      row(dw_bn_beta), row(dw_bn_mean), row(dw_bn_var),
      point_w, row(point_b), row(point_bn_gamma), row(point_bn_beta),
      row(point_bn_mean), row(point_bn_var),
      shortcut_w, row(shortcut_b), row(shortcut_bn_gamma),
      row(shortcut_bn_beta), row(shortcut_bn_mean), row(shortcut_bn_var),
      se_w1, row(se_b1), se_w2, row(se_b2))

    return out.reshape(N, Cout, H, W)
